# baseline (speedup 1.0000x reference)
import functools

import jax
import jax.numpy as jnp
from jax import lax
from jax.experimental import pallas as pl
from jax.experimental.pallas import tpu as pltpu
from jax.experimental.pallas import tpu_sc as plsc

N = 50000
B = 1024
D = 512
HS = 4096
EPS = 1e-5

NC = 2
NS = 16
NW = NC * NS
SEG_PER_W = B // NW
CH = 96
EFF = CH - 8
LANES = 16


def _pool_body(x_hbm, bat_hbm, pooled_hbm, acc, offs_s, sem0, sem1):
    wid = lax.axis_index("s") * NC + lax.axis_index("c")
    seg_lo = wid * SEG_PER_W

    def phase1(bat):
        pltpu.sync_copy(bat_hbm, bat.at[pl.ds(0, N)])

        def bat_at(i):
            return bat[pl.ds(i, LANES)][0]

        for j in range(SEG_PER_W + 1):
            target = seg_lo + j

            def bs_step(_, lohi, target=target):
                lo, hi = lohi
                active = lo < hi
                mid = jnp.minimum((lo + hi) >> 1, N - 1)
                v = bat_at(mid)
                go_right = active & (v < target)
                lo = jnp.where(go_right, mid + 1, lo)
                hi = jnp.where(active & jnp.logical_not(v < target), mid, hi)
                return lo, hi

            lo, _ = lax.fori_loop(0, 17, bs_step, (jnp.int32(0), jnp.int32(N)))
            offs_s[j] = lo

    pl.run_scoped(phase1, pltpu.VMEM((N + LANES,), jnp.int32))

    r_lo = offs_s[0]
    r_hi = offs_s[SEG_PER_W]
    nch = (r_hi - r_lo + EFF - 1) // EFF

    def zero_body(j, _):
        def zrow(k, _):
            acc[j, pl.ds(k * LANES, LANES)] = jnp.zeros((LANES,), jnp.float32)
            return 0

        lax.fori_loop(0, D // LANES, zrow, 0)
        return 0

    lax.fori_loop(0, SEG_PER_W, zero_body, 0)

    def phase2(xb0, xb1):
        def dma(c, buf, sem):
            r0 = r_lo + c * EFF
            r0a = pl.multiple_of(jnp.minimum((r0 >> 3) << 3, N - CH), 8)
            return pltpu.make_async_copy(x_hbm.at[pl.ds(r0a, CH)], buf, sem)

        def process(c, buf):
            r0 = r_lo + c * EFF
            r0a = pl.multiple_of(jnp.minimum((r0 >> 3) << 3, N - CH), 8)
            hi_c = jnp.minimum(r0 + EFF, r_hi)

            def seg_body(j, _):
                lo = jnp.maximum(offs_s[j], r0) - r0a
                hi = jnp.minimum(offs_s[j + 1], hi_c) - r0a

                @pl.when(hi > lo)
                def _():
                    def k_body(k, _):
                        kb = k * (4 * LANES)
                        z = jnp.zeros((LANES,), jnp.float32)

                        def r_body(i, accs):
                            a0, a1, a2, a3 = accs
                            return (
                                a0 + buf[i, pl.ds(kb, LANES)],
                                a1 + buf[i, pl.ds(kb + LANES, LANES)],
                                a2 + buf[i, pl.ds(kb + 2 * LANES, LANES)],
                                a3 + buf[i, pl.ds(kb + 3 * LANES, LANES)],
                            )

                        a0, a1, a2, a3 = lax.fori_loop(lo, hi, r_body, (z, z, z, z))
                        plsc.addupdate(acc.at[j, pl.ds(kb, LANES)], a0)
                        plsc.addupdate(acc.at[j, pl.ds(kb + LANES, LANES)], a1)
                        plsc.addupdate(acc.at[j, pl.ds(kb + 2 * LANES, LANES)], a2)
                        plsc.addupdate(acc.at[j, pl.ds(kb + 3 * LANES, LANES)], a3)
                        return 0

                    lax.fori_loop(0, D // (4 * LANES), k_body, 0)

                return 0

            lax.fori_loop(0, SEG_PER_W, seg_body, 0)

        @pl.when(nch > 0)
        def _():
            dma(0, xb0, sem0).start()

        def pair_body(p, _):
            c0 = 2 * p
            c1 = c0 + 1

            @pl.when(c1 < nch)
            def _():
                dma(c1, xb1, sem1).start()

            dma(c0, xb0, sem0).wait()
            process(c0, xb0)

            @pl.when(c1 + 1 < nch)
            def _():
                dma(c1 + 1, xb0, sem0).start()

            @pl.when(c1 < nch)
            def _():
                dma(c1, xb1, sem1).wait()
                process(c1, xb1)

            return 0

        lax.fori_loop(0, (nch + 1) // 2, pair_body, 0)

    pl.run_scoped(
        phase2,
        pltpu.VMEM((CH, D), jnp.float32),
        pltpu.VMEM((CH, D), jnp.float32),
    )

    for j in range(SEG_PER_W):
        cnt = (offs_s[j + 1] - offs_s[j]).astype(jnp.float32)
        cntv = jnp.full((LANES,), cnt, jnp.float32)
        inv = jnp.ones((LANES,), jnp.float32) / jnp.maximum(cntv, 1.0)

        def scale_body(k, _, j=j, inv=inv):
            sl = pl.ds(k * LANES, LANES)
            acc[j, sl] = acc[j, sl] * inv
            return 0

        lax.fori_loop(0, D // LANES, scale_body, 0)

    pltpu.sync_copy(acc, pooled_hbm.at[pl.ds(seg_lo, SEG_PER_W)])


@functools.lru_cache(maxsize=None)
def _pool_fn():
    return pl.kernel(
        _pool_body,
        out_type=jax.ShapeDtypeStruct((B, D), jnp.float32),
        mesh=plsc.VectorSubcoreMesh(
            core_axis_name="c", subcore_axis_name="s", num_cores=NC, num_subcores=NS
        ),
        scratch_types=[
            pltpu.VMEM((SEG_PER_W, D), jnp.float32),
            pltpu.SMEM((SEG_PER_W + 1,), jnp.int32),
            pltpu.SemaphoreType.DMA,
            pltpu.SemaphoreType.DMA,
        ],
    )


TJ1 = 512


_DN_NT = (((1,), (1,)), ((), ()))


def _mlp1a_body(u_ref, w1_ref, b1_ref, o_ref):
    h = lax.dot_general(u_ref[...].astype(jnp.bfloat16),
                        w1_ref[:, :D].astype(jnp.bfloat16), _DN_NT,
                        preferred_element_type=jnp.float32)
    o_ref[...] = h + b1_ref[...]


def _mlp1a(u, w1, b1):
    return pl.pallas_call(
        _mlp1a_body,
        grid=(HS // TJ1,),
        in_specs=[
            pl.BlockSpec((B, D), lambda j: (0, 0)),
            pl.BlockSpec((TJ1, 2 * D), lambda j: (j, 0)),
            pl.BlockSpec((1, TJ1), lambda j: (0, j)),
        ],
        out_specs=pl.BlockSpec((B, TJ1), lambda j: (0, j)),
        out_shape=jax.ShapeDtypeStruct((B, HS), jnp.float32),
        compiler_params=pltpu.CompilerParams(
            dimension_semantics=("arbitrary",),
        ),
    )(u, w1, b1)


def _mlp1b_body(hu_ref, p_ref, w1_ref, g1_ref, be1_ref, o_ref):
    h = hu_ref[...] + lax.dot_general(
        p_ref[...].astype(jnp.bfloat16), w1_ref[:, D:].astype(jnp.bfloat16),
        _DN_NT, preferred_element_type=jnp.float32)
    mean = jnp.mean(h, axis=0, keepdims=True)
    var = jnp.mean((h - mean) * (h - mean), axis=0, keepdims=True)
    hn = (h - mean) * lax.rsqrt(var + EPS) * g1_ref[...] + be1_ref[...]
    o_ref[...] = jnp.maximum(hn, 0.0).astype(jnp.bfloat16)


def _mlp1b(hu, pooled, w1, g1, be1):
    return pl.pallas_call(
        _mlp1b_body,
        grid=(HS // TJ1,),
        in_specs=[
            pl.BlockSpec((B, TJ1), lambda j: (0, j)),
            pl.BlockSpec((B, D), lambda j: (0, 0)),
            pl.BlockSpec((TJ1, 2 * D), lambda j: (j, 0)),
            pl.BlockSpec((1, TJ1), lambda j: (0, j)),
            pl.BlockSpec((1, TJ1), lambda j: (0, j)),
        ],
        out_specs=pl.BlockSpec((B, TJ1), lambda j: (0, j)),
        out_shape=jax.ShapeDtypeStruct((B, HS), jnp.bfloat16),
        compiler_params=pltpu.CompilerParams(
            dimension_semantics=("arbitrary",),
        ),
    )(hu, pooled, w1, g1, be1)


TJ2 = 512


def _mlp2_body(a_ref, w2_ref, b2_ref, g2_ref, be2_ref, o_ref):
    h = lax.dot_general(a_ref[...], w2_ref[...].astype(jnp.bfloat16),
                        _DN_NT, preferred_element_type=jnp.float32)
    h = h + b2_ref[...]
    mean = jnp.mean(h, axis=0, keepdims=True)
    var = jnp.mean((h - mean) * (h - mean), axis=0, keepdims=True)
    o_ref[...] = (h - mean) * lax.rsqrt(var + EPS) * g2_ref[...] + be2_ref[...]


def _mlp2(a1, w2, b2, g2, be2):
    return pl.pallas_call(
        _mlp2_body,
        grid=(HS // TJ2,),
        in_specs=[
            pl.BlockSpec((B, HS), lambda j: (0, 0)),
            pl.BlockSpec((TJ2, HS), lambda j: (j, 0)),
            pl.BlockSpec((1, TJ2), lambda j: (0, j)),
            pl.BlockSpec((1, TJ2), lambda j: (0, j)),
            pl.BlockSpec((1, TJ2), lambda j: (0, j)),
        ],
        out_specs=pl.BlockSpec((B, TJ2), lambda j: (0, j)),
        out_shape=jax.ShapeDtypeStruct((B, HS), jnp.float32),
        compiler_params=pltpu.CompilerParams(
            dimension_semantics=("arbitrary",),
        ),
    )(a1, w2, b2, g2, be2)


def kernel(x, u, batch, W1, b1, g1, be1, W2, b2, g2, be2):
    bat = batch.astype(jnp.int32)
    hu = _mlp1a(u, W1, b1.reshape(1, HS))
    pooled = _pool_fn()(x, bat)
    a1 = _mlp1b(hu, pooled, W1, g1.reshape(1, HS), be1.reshape(1, HS))
    return _mlp2(
        a1, W2, b2.reshape(1, HS), g2.reshape(1, HS), be2.reshape(1, HS)
    )


# speedup vs baseline: 1.0064x; 1.0064x over previous
import functools

import jax
import jax.numpy as jnp
from jax import lax
from jax.experimental import pallas as pl
from jax.experimental.pallas import tpu as pltpu
from jax.experimental.pallas import tpu_sc as plsc

N = 50000
B = 1024
D = 512
HS = 4096
EPS = 1e-5

NC = 2
NS = 16
NW = NC * NS
SEG_PER_W = B // NW
CH = 64
EFF = CH - 8
LANES = 16


def _pool_body(x_hbm, bat_hbm, pooled_hbm, acc, offs_v, sem0, sem1, sem2):
    wid = lax.axis_index("s") * NC + lax.axis_index("c")
    seg_lo = wid * SEG_PER_W

    def offs_at(j):
        return offs_v[j]

    def phase1(bat):
        pltpu.sync_copy(bat_hbm, bat.at[pl.ds(0, N)])

        def bat_at(i):
            return bat[pl.ds(i, LANES)][0]

        for j in range(SEG_PER_W + 1):
            target = seg_lo + j

            def bs_step(_, lohi, target=target):
                lo, hi = lohi
                active = lo < hi
                mid = jnp.minimum((lo + hi) >> 1, N - 1)
                v = bat_at(mid)
                go_right = active & (v < target)
                lo = jnp.where(go_right, mid + 1, lo)
                hi = jnp.where(active & jnp.logical_not(v < target), mid, hi)
                return lo, hi

            lo, _ = lax.fori_loop(0, 17, bs_step, (jnp.int32(0), jnp.int32(N)))
            offs_v[j] = lo

    pl.run_scoped(phase1, pltpu.VMEM((N + LANES,), jnp.int32))

    r_lo = offs_at(0)
    r_hi = offs_at(SEG_PER_W)
    nch = (r_hi - r_lo + EFF - 1) // EFF

    def zero_body(j, _):
        def zrow(k, _):
            acc[j, pl.ds(k * LANES, LANES)] = jnp.zeros((LANES,), jnp.float32)
            return 0

        lax.fori_loop(0, D // LANES, zrow, 0)
        return 0

    lax.fori_loop(0, SEG_PER_W, zero_body, 0)

    def phase2(xb0, xb1, xb2):
        def dma(c, buf, sem):
            r0 = r_lo + c * EFF
            r0a = pl.multiple_of(jnp.minimum((r0 >> 3) << 3, N - CH), 8)
            return pltpu.make_async_copy(x_hbm.at[pl.ds(r0a, CH)], buf, sem)

        def process(c, buf):
            r0 = r_lo + c * EFF
            r0a = pl.multiple_of(jnp.minimum((r0 >> 3) << 3, N - CH), 8)
            hi_c = jnp.minimum(r0 + EFF, r_hi)

            def seg_body(j, _):
                lo = jnp.maximum(offs_at(j), r0) - r0a
                hi = jnp.minimum(offs_at(j + 1), hi_c) - r0a

                @pl.when(hi > lo)
                def _():
                    def k_body(k, _):
                        kb = k * (4 * LANES)
                        z = jnp.zeros((LANES,), jnp.float32)

                        def r_body(i, accs):
                            a0, a1, a2, a3 = accs
                            return (
                                a0 + buf[i, pl.ds(kb, LANES)],
                                a1 + buf[i, pl.ds(kb + LANES, LANES)],
                                a2 + buf[i, pl.ds(kb + 2 * LANES, LANES)],
                                a3 + buf[i, pl.ds(kb + 3 * LANES, LANES)],
                            )

                        a0, a1, a2, a3 = lax.fori_loop(lo, hi, r_body, (z, z, z, z))
                        plsc.addupdate(acc.at[j, pl.ds(kb, LANES)], a0)
                        plsc.addupdate(acc.at[j, pl.ds(kb + LANES, LANES)], a1)
                        plsc.addupdate(acc.at[j, pl.ds(kb + 2 * LANES, LANES)], a2)
                        plsc.addupdate(acc.at[j, pl.ds(kb + 3 * LANES, LANES)], a3)
                        return 0

                    lax.fori_loop(0, D // (4 * LANES), k_body, 0)

                return 0

            lax.fori_loop(0, SEG_PER_W, seg_body, 0)

        bufs = (xb0, xb1, xb2)
        sems = (sem0, sem1, sem2)

        @pl.when(nch > 0)
        def _():
            dma(0, xb0, sem0).start()

        @pl.when(nch > 1)
        def _():
            dma(1, xb1, sem1).start()

        def tri_body(t, _):
            for q in range(3):
                c = 3 * t + q
                nxt = c + 2

                @pl.when(nxt < nch)
                def _(c=c, q=q, nxt=nxt):
                    dma(nxt, bufs[(q + 2) % 3], sems[(q + 2) % 3]).start()

                @pl.when(c < nch)
                def _(c=c, q=q):
                    dma(c, bufs[q], sems[q]).wait()
                    process(c, bufs[q])

            return 0

        lax.fori_loop(0, (nch + 2) // 3, tri_body, 0)

    pl.run_scoped(
        phase2,
        pltpu.VMEM((CH, D), jnp.float32),
        pltpu.VMEM((CH, D), jnp.float32),
        pltpu.VMEM((CH, D), jnp.float32),
    )

    for j in range(SEG_PER_W):
        cnt = (offs_at(j + 1) - offs_at(j)).astype(jnp.float32)
        cntv = jnp.full((LANES,), cnt, jnp.float32)
        inv = jnp.ones((LANES,), jnp.float32) / jnp.maximum(cntv, 1.0)

        def scale_body(k, _, j=j, inv=inv):
            sl = pl.ds(k * LANES, LANES)
            acc[j, sl] = acc[j, sl] * inv
            return 0

        lax.fori_loop(0, D // LANES, scale_body, 0)

    pltpu.sync_copy(acc, pooled_hbm.at[pl.ds(seg_lo, SEG_PER_W)])


@functools.lru_cache(maxsize=None)
def _pool_fn():
    return pl.kernel(
        _pool_body,
        out_type=jax.ShapeDtypeStruct((B, D), jnp.float32),
        mesh=plsc.VectorSubcoreMesh(
            core_axis_name="c", subcore_axis_name="s", num_cores=NC, num_subcores=NS
        ),
        scratch_types=[
            pltpu.VMEM((SEG_PER_W, D), jnp.float32),
            pltpu.SMEM((SEG_PER_W + 1,), jnp.int32),
            pltpu.SemaphoreType.DMA,
            pltpu.SemaphoreType.DMA,
            pltpu.SemaphoreType.DMA,
        ],
    )


TJ1 = 512


_DN_NT = (((1,), (1,)), ((), ()))


def _mlp1_body(u_ref, p_ref, w1_ref, b1_ref, g1_ref, be1_ref, o_ref):
    w1 = w1_ref[...].astype(jnp.bfloat16)
    h = lax.dot_general(u_ref[...].astype(jnp.bfloat16), w1[:, :D], _DN_NT,
                        preferred_element_type=jnp.float32)
    h = h + lax.dot_general(p_ref[...].astype(jnp.bfloat16), w1[:, D:], _DN_NT,
                            preferred_element_type=jnp.float32)
    h = h + b1_ref[...]
    mean = jnp.mean(h, axis=0, keepdims=True)
    var = jnp.mean((h - mean) * (h - mean), axis=0, keepdims=True)
    hn = (h - mean) * lax.rsqrt(var + EPS) * g1_ref[...] + be1_ref[...]
    o_ref[...] = jnp.maximum(hn, 0.0).astype(jnp.bfloat16)


def _mlp1(u, pooled, w1, b1, g1, be1):
    return pl.pallas_call(
        _mlp1_body,
        grid=(HS // TJ1,),
        in_specs=[
            pl.BlockSpec((B, D), lambda j: (0, 0)),
            pl.BlockSpec((B, D), lambda j: (0, 0)),
            pl.BlockSpec((TJ1, 2 * D), lambda j: (j, 0)),
            pl.BlockSpec((1, TJ1), lambda j: (0, j)),
            pl.BlockSpec((1, TJ1), lambda j: (0, j)),
            pl.BlockSpec((1, TJ1), lambda j: (0, j)),
        ],
        out_specs=pl.BlockSpec((B, TJ1), lambda j: (0, j)),
        out_shape=jax.ShapeDtypeStruct((B, HS), jnp.bfloat16),
        compiler_params=pltpu.CompilerParams(
            dimension_semantics=("arbitrary",),
        ),
    )(u, pooled, w1, b1, g1, be1)


TJ2 = 512


def _mlp2_body(a_ref, w2_ref, b2_ref, g2_ref, be2_ref, o_ref):
    h = lax.dot_general(a_ref[...], w2_ref[...].astype(jnp.bfloat16),
                        _DN_NT, preferred_element_type=jnp.float32)
    h = h + b2_ref[...]
    mean = jnp.mean(h, axis=0, keepdims=True)
    var = jnp.mean((h - mean) * (h - mean), axis=0, keepdims=True)
    o_ref[...] = (h - mean) * lax.rsqrt(var + EPS) * g2_ref[...] + be2_ref[...]


def _mlp2(a1, w2, b2, g2, be2):
    return pl.pallas_call(
        _mlp2_body,
        grid=(HS // TJ2,),
        in_specs=[
            pl.BlockSpec((B, HS), lambda j: (0, 0)),
            pl.BlockSpec((TJ2, HS), lambda j: (j, 0)),
            pl.BlockSpec((1, TJ2), lambda j: (0, j)),
            pl.BlockSpec((1, TJ2), lambda j: (0, j)),
            pl.BlockSpec((1, TJ2), lambda j: (0, j)),
        ],
        out_specs=pl.BlockSpec((B, TJ2), lambda j: (0, j)),
        out_shape=jax.ShapeDtypeStruct((B, HS), jnp.float32),
        compiler_params=pltpu.CompilerParams(
            dimension_semantics=("arbitrary",),
        ),
    )(a1, w2, b2, g2, be2)


def kernel(x, u, batch, W1, b1, g1, be1, W2, b2, g2, be2):
    bat = batch.astype(jnp.int32)
    pooled = _pool_fn()(x, bat)
    a1 = _mlp1(
        u, pooled, W1,
        b1.reshape(1, HS), g1.reshape(1, HS), be1.reshape(1, HS),
    )
    return _mlp2(
        a1, W2, b2.reshape(1, HS), g2.reshape(1, HS), be2.reshape(1, HS)
    )


# speedup vs baseline: 1.0338x; 1.0271x over previous
import functools

import jax
import jax.numpy as jnp
from jax import lax
from jax.experimental import pallas as pl
from jax.experimental.pallas import tpu as pltpu
from jax.experimental.pallas import tpu_sc as plsc

N = 50000
B = 1024
D = 512
HS = 4096
EPS = 1e-5

NC = 2
NS = 16
NW = NC * NS
SEG_PER_W = B // NW
CH = 96
EFF = CH - 8
LANES = 16


def _pool_body(x_hbm, bat_hbm, pooled_hbm, acc, offs_v, sem0, sem1, sem2):
    wid = lax.axis_index("s") * NC + lax.axis_index("c")
    seg_lo = wid * SEG_PER_W

    def offs_at(j):
        return offs_v[j]

    def phase1(bat):
        pltpu.sync_copy(bat_hbm, bat.at[pl.ds(0, N)])

        def bat_at(i):
            return bat[pl.ds(i, LANES)][0]

        for j in range(SEG_PER_W + 1):
            target = seg_lo + j

            def bs_step(_, lohi, target=target):
                lo, hi = lohi
                active = lo < hi
                mid = jnp.minimum((lo + hi) >> 1, N - 1)
                v = bat_at(mid)
                go_right = active & (v < target)
                lo = jnp.where(go_right, mid + 1, lo)
                hi = jnp.where(active & jnp.logical_not(v < target), mid, hi)
                return lo, hi

            lo, _ = lax.fori_loop(0, 17, bs_step, (jnp.int32(0), jnp.int32(N)))
            offs_v[j] = lo

    pl.run_scoped(phase1, pltpu.VMEM((N + LANES,), jnp.int32))

    r_lo = offs_at(0)
    r_hi = offs_at(SEG_PER_W)
    nch = (r_hi - r_lo + EFF - 1) // EFF

    def zero_body(j, _):
        def zrow(k, _):
            acc[j, pl.ds(k * LANES, LANES)] = jnp.zeros((LANES,), jnp.float32)
            return 0

        lax.fori_loop(0, D // LANES, zrow, 0)
        return 0

    lax.fori_loop(0, SEG_PER_W, zero_body, 0)

    def phase2(xb0, xb1):
        def dma(c, buf, sem):
            r0 = r_lo + c * EFF
            r0a = pl.multiple_of(jnp.minimum((r0 >> 3) << 3, N - CH), 8)
            return pltpu.make_async_copy(x_hbm.at[pl.ds(r0a, CH)], buf, sem)

        def process(c, buf):
            r0 = r_lo + c * EFF
            r0a = pl.multiple_of(jnp.minimum((r0 >> 3) << 3, N - CH), 8)
            hi_c = jnp.minimum(r0 + EFF, r_hi)

            def seg_body(j, _):
                lo = jnp.maximum(offs_at(j), r0) - r0a
                hi = jnp.minimum(offs_at(j + 1), hi_c) - r0a

                @pl.when(hi > lo)
                def _():
                    def k_body(k, _):
                        kb = k * (4 * LANES)
                        z = jnp.zeros((LANES,), jnp.float32)

                        @plsc.parallel_loop(lo, hi, unroll=4, carry=(z, z, z, z))
                        def r_body(i, accs):
                            a0, a1, a2, a3 = accs
                            return (
                                a0 + buf[i, pl.ds(kb, LANES)],
                                a1 + buf[i, pl.ds(kb + LANES, LANES)],
                                a2 + buf[i, pl.ds(kb + 2 * LANES, LANES)],
                                a3 + buf[i, pl.ds(kb + 3 * LANES, LANES)],
                            )

                        a0, a1, a2, a3 = r_body
                        plsc.addupdate(acc.at[j, pl.ds(kb, LANES)], a0)
                        plsc.addupdate(acc.at[j, pl.ds(kb + LANES, LANES)], a1)
                        plsc.addupdate(acc.at[j, pl.ds(kb + 2 * LANES, LANES)], a2)
                        plsc.addupdate(acc.at[j, pl.ds(kb + 3 * LANES, LANES)], a3)
                        return 0

                    lax.fori_loop(0, D // (4 * LANES), k_body, 0)

                return 0

            lax.fori_loop(0, SEG_PER_W, seg_body, 0)

        @pl.when(nch > 0)
        def _():
            dma(0, xb0, sem0).start()

        def pair_body(p, _):
            c0 = 2 * p
            c1 = c0 + 1

            @pl.when(c1 < nch)
            def _():
                dma(c1, xb1, sem1).start()

            dma(c0, xb0, sem0).wait()
            process(c0, xb0)

            @pl.when(c1 + 1 < nch)
            def _():
                dma(c1 + 1, xb0, sem0).start()

            @pl.when(c1 < nch)
            def _():
                dma(c1, xb1, sem1).wait()
                process(c1, xb1)

            return 0

        lax.fori_loop(0, (nch + 1) // 2, pair_body, 0)

    pl.run_scoped(
        phase2,
        pltpu.VMEM((CH, D), jnp.float32),
        pltpu.VMEM((CH, D), jnp.float32),
    )

    for j in range(SEG_PER_W):
        cnt = (offs_at(j + 1) - offs_at(j)).astype(jnp.float32)
        cntv = jnp.full((LANES,), cnt, jnp.float32)
        inv = jnp.ones((LANES,), jnp.float32) / jnp.maximum(cntv, 1.0)

        def scale_body(k, _, j=j, inv=inv):
            sl = pl.ds(k * LANES, LANES)
            acc[j, sl] = acc[j, sl] * inv
            return 0

        lax.fori_loop(0, D // LANES, scale_body, 0)

    pltpu.sync_copy(acc, pooled_hbm.at[pl.ds(seg_lo, SEG_PER_W)])


@functools.lru_cache(maxsize=None)
def _pool_fn():
    return pl.kernel(
        _pool_body,
        out_type=jax.ShapeDtypeStruct((B, D), jnp.float32),
        mesh=plsc.VectorSubcoreMesh(
            core_axis_name="c", subcore_axis_name="s", num_cores=NC, num_subcores=NS
        ),
        scratch_types=[
            pltpu.VMEM((SEG_PER_W, D), jnp.float32),
            pltpu.SMEM((SEG_PER_W + 1,), jnp.int32),
            pltpu.SemaphoreType.DMA,
            pltpu.SemaphoreType.DMA,
            pltpu.SemaphoreType.DMA,
        ],
    )


TJ1 = 512


_DN_NT = (((1,), (1,)), ((), ()))


def _mlp1_body(u_ref, p_ref, w1_ref, b1_ref, g1_ref, be1_ref, o_ref):
    w1 = w1_ref[...].astype(jnp.bfloat16)
    h = lax.dot_general(u_ref[...].astype(jnp.bfloat16), w1[:, :D], _DN_NT,
                        preferred_element_type=jnp.float32)
    h = h + lax.dot_general(p_ref[...].astype(jnp.bfloat16), w1[:, D:], _DN_NT,
                            preferred_element_type=jnp.float32)
    h = h + b1_ref[...]
    mean = jnp.mean(h, axis=0, keepdims=True)
    var = jnp.mean((h - mean) * (h - mean), axis=0, keepdims=True)
    hn = (h - mean) * lax.rsqrt(var + EPS) * g1_ref[...] + be1_ref[...]
    o_ref[...] = jnp.maximum(hn, 0.0).astype(jnp.bfloat16)


def _mlp1(u, pooled, w1, b1, g1, be1):
    return pl.pallas_call(
        _mlp1_body,
        grid=(HS // TJ1,),
        in_specs=[
            pl.BlockSpec((B, D), lambda j: (0, 0)),
            pl.BlockSpec((B, D), lambda j: (0, 0)),
            pl.BlockSpec((TJ1, 2 * D), lambda j: (j, 0)),
            pl.BlockSpec((1, TJ1), lambda j: (0, j)),
            pl.BlockSpec((1, TJ1), lambda j: (0, j)),
            pl.BlockSpec((1, TJ1), lambda j: (0, j)),
        ],
        out_specs=pl.BlockSpec((B, TJ1), lambda j: (0, j)),
        out_shape=jax.ShapeDtypeStruct((B, HS), jnp.bfloat16),
        compiler_params=pltpu.CompilerParams(
            dimension_semantics=("arbitrary",),
        ),
    )(u, pooled, w1, b1, g1, be1)


TJ2 = 512


def _mlp2_body(a_ref, w2_ref, b2_ref, g2_ref, be2_ref, o_ref):
    h = lax.dot_general(a_ref[...], w2_ref[...].astype(jnp.bfloat16),
                        _DN_NT, preferred_element_type=jnp.float32)
    h = h + b2_ref[...]
    mean = jnp.mean(h, axis=0, keepdims=True)
    var = jnp.mean((h - mean) * (h - mean), axis=0, keepdims=True)
    o_ref[...] = (h - mean) * lax.rsqrt(var + EPS) * g2_ref[...] + be2_ref[...]


def _mlp2(a1, w2, b2, g2, be2):
    return pl.pallas_call(
        _mlp2_body,
        grid=(HS // TJ2,),
        in_specs=[
            pl.BlockSpec((B, HS), lambda j: (0, 0)),
            pl.BlockSpec((TJ2, HS), lambda j: (j, 0)),
            pl.BlockSpec((1, TJ2), lambda j: (0, j)),
            pl.BlockSpec((1, TJ2), lambda j: (0, j)),
            pl.BlockSpec((1, TJ2), lambda j: (0, j)),
        ],
        out_specs=pl.BlockSpec((B, TJ2), lambda j: (0, j)),
        out_shape=jax.ShapeDtypeStruct((B, HS), jnp.float32),
        compiler_params=pltpu.CompilerParams(
            dimension_semantics=("arbitrary",),
        ),
    )(a1, w2, b2, g2, be2)


def kernel(x, u, batch, W1, b1, g1, be1, W2, b2, g2, be2):
    bat = batch.astype(jnp.int32)
    pooled = _pool_fn()(x, bat)
    a1 = _mlp1(
        u, pooled, W1,
        b1.reshape(1, HS), g1.reshape(1, HS), be1.reshape(1, HS),
    )
    return _mlp2(
        a1, W2, b2.reshape(1, HS), g2.reshape(1, HS), be2.reshape(1, HS)
    )


# speedup vs baseline: 1.1450x; 1.1076x over previous
import functools

import jax
import jax.numpy as jnp
from jax import lax
from jax.experimental import pallas as pl
from jax.experimental.pallas import tpu as pltpu
from jax.experimental.pallas import tpu_sc as plsc

N = 50000
B = 1024
D = 512
HS = 4096
EPS = 1e-5

NC = 2
NS = 16
NW = NC * NS
SEG_PER_W = B // NW
CH = 96
EFF = CH - 8
LANES = 16


def _pool_body(x_hbm, bat_hbm, pooled_hbm, acc, offs_v, sem0, sem1, sem2):
    wid = lax.axis_index("s") * NC + lax.axis_index("c")
    seg_lo = wid * SEG_PER_W

    def offs_at(j):
        return offs_v[j]

    def phase1(bat):
        pltpu.sync_copy(bat_hbm, bat.at[pl.ds(0, N)])

        def bat_at(i):
            return bat[pl.ds(i, LANES)][0]

        for j in range(SEG_PER_W + 1):
            target = seg_lo + j

            def bs_step(_, lohi, target=target):
                lo, hi = lohi
                active = lo < hi
                mid = jnp.minimum((lo + hi) >> 1, N - 1)
                v = bat_at(mid)
                go_right = active & (v < target)
                lo = jnp.where(go_right, mid + 1, lo)
                hi = jnp.where(active & jnp.logical_not(v < target), mid, hi)
                return lo, hi

            lo, _ = lax.fori_loop(0, 17, bs_step, (jnp.int32(0), jnp.int32(N)))
            offs_v[j] = lo

    pl.run_scoped(phase1, pltpu.VMEM((N + LANES,), jnp.int32))

    r_lo = offs_at(0)
    r_hi = offs_at(SEG_PER_W)
    nch = (r_hi - r_lo + EFF - 1) // EFF

    def zero_body(j, _):
        def zrow(k, _):
            acc[j, pl.ds(k * LANES, LANES)] = jnp.zeros((LANES,), jnp.float32)
            return 0

        lax.fori_loop(0, D // LANES, zrow, 0)
        return 0

    lax.fori_loop(0, SEG_PER_W, zero_body, 0)

    def phase2(xb0, xb1):
        def dma(c, buf, sem):
            r0 = r_lo + c * EFF
            r0a = pl.multiple_of(jnp.minimum((r0 >> 3) << 3, N - CH), 8)
            return pltpu.make_async_copy(x_hbm.at[pl.ds(r0a, CH)], buf, sem)

        def process(c, buf):
            r0 = r_lo + c * EFF
            r0a = pl.multiple_of(jnp.minimum((r0 >> 3) << 3, N - CH), 8)
            hi_c = jnp.minimum(r0 + EFF, r_hi)

            def seg_body(j, _):
                lo = jnp.maximum(offs_at(j), r0) - r0a
                hi = jnp.minimum(offs_at(j + 1), hi_c) - r0a

                @pl.when(hi > lo)
                def _():
                    z = jnp.zeros((LANES,), jnp.float32)

                    def r_body(i, accs):
                        return tuple(
                            accs[k] + buf[i, pl.ds(k * LANES, LANES)]
                            for k in range(D // LANES)
                        )

                    accs = lax.fori_loop(lo, hi, r_body, (z,) * (D // LANES))
                    for k in range(D // LANES):
                        plsc.addupdate(acc.at[j, pl.ds(k * LANES, LANES)], accs[k])

                return 0

            lax.fori_loop(0, SEG_PER_W, seg_body, 0)

        @pl.when(nch > 0)
        def _():
            dma(0, xb0, sem0).start()

        def pair_body(p, _):
            c0 = 2 * p
            c1 = c0 + 1

            @pl.when(c1 < nch)
            def _():
                dma(c1, xb1, sem1).start()

            dma(c0, xb0, sem0).wait()
            process(c0, xb0)

            @pl.when(c1 + 1 < nch)
            def _():
                dma(c1 + 1, xb0, sem0).start()

            @pl.when(c1 < nch)
            def _():
                dma(c1, xb1, sem1).wait()
                process(c1, xb1)

            return 0

        lax.fori_loop(0, (nch + 1) // 2, pair_body, 0)

    pl.run_scoped(
        phase2,
        pltpu.VMEM((CH, D), jnp.float32),
        pltpu.VMEM((CH, D), jnp.float32),
    )

    for j in range(SEG_PER_W):
        cnt = (offs_at(j + 1) - offs_at(j)).astype(jnp.float32)
        cntv = jnp.full((LANES,), cnt, jnp.float32)
        inv = jnp.ones((LANES,), jnp.float32) / jnp.maximum(cntv, 1.0)

        def scale_body(k, _, j=j, inv=inv):
            sl = pl.ds(k * LANES, LANES)
            acc[j, sl] = acc[j, sl] * inv
            return 0

        lax.fori_loop(0, D // LANES, scale_body, 0)

    pltpu.sync_copy(acc, pooled_hbm.at[pl.ds(seg_lo, SEG_PER_W)])


@functools.lru_cache(maxsize=None)
def _pool_fn():
    return pl.kernel(
        _pool_body,
        out_type=jax.ShapeDtypeStruct((B, D), jnp.float32),
        mesh=plsc.VectorSubcoreMesh(
            core_axis_name="c", subcore_axis_name="s", num_cores=NC, num_subcores=NS
        ),
        scratch_types=[
            pltpu.VMEM((SEG_PER_W, D), jnp.float32),
            pltpu.SMEM((SEG_PER_W + 1,), jnp.int32),
            pltpu.SemaphoreType.DMA,
            pltpu.SemaphoreType.DMA,
            pltpu.SemaphoreType.DMA,
        ],
    )


TJ1 = 512


_DN_NT = (((1,), (1,)), ((), ()))


def _mlp1_body(u_ref, p_ref, w1_ref, b1_ref, g1_ref, be1_ref, o_ref):
    w1 = w1_ref[...].astype(jnp.bfloat16)
    h = lax.dot_general(u_ref[...].astype(jnp.bfloat16), w1[:, :D], _DN_NT,
                        preferred_element_type=jnp.float32)
    h = h + lax.dot_general(p_ref[...].astype(jnp.bfloat16), w1[:, D:], _DN_NT,
                            preferred_element_type=jnp.float32)
    h = h + b1_ref[...]
    mean = jnp.mean(h, axis=0, keepdims=True)
    var = jnp.mean((h - mean) * (h - mean), axis=0, keepdims=True)
    hn = (h - mean) * lax.rsqrt(var + EPS) * g1_ref[...] + be1_ref[...]
    o_ref[...] = jnp.maximum(hn, 0.0).astype(jnp.bfloat16)


def _mlp1(u, pooled, w1, b1, g1, be1):
    return pl.pallas_call(
        _mlp1_body,
        grid=(HS // TJ1,),
        in_specs=[
            pl.BlockSpec((B, D), lambda j: (0, 0)),
            pl.BlockSpec((B, D), lambda j: (0, 0)),
            pl.BlockSpec((TJ1, 2 * D), lambda j: (j, 0)),
            pl.BlockSpec((1, TJ1), lambda j: (0, j)),
            pl.BlockSpec((1, TJ1), lambda j: (0, j)),
            pl.BlockSpec((1, TJ1), lambda j: (0, j)),
        ],
        out_specs=pl.BlockSpec((B, TJ1), lambda j: (0, j)),
        out_shape=jax.ShapeDtypeStruct((B, HS), jnp.bfloat16),
        compiler_params=pltpu.CompilerParams(
            dimension_semantics=("arbitrary",),
        ),
    )(u, pooled, w1, b1, g1, be1)


TJ2 = 512


def _mlp2_body(a_ref, w2_ref, b2_ref, g2_ref, be2_ref, o_ref):
    h = lax.dot_general(a_ref[...], w2_ref[...].astype(jnp.bfloat16),
                        _DN_NT, preferred_element_type=jnp.float32)
    h = h + b2_ref[...]
    mean = jnp.mean(h, axis=0, keepdims=True)
    var = jnp.mean((h - mean) * (h - mean), axis=0, keepdims=True)
    o_ref[...] = (h - mean) * lax.rsqrt(var + EPS) * g2_ref[...] + be2_ref[...]


def _mlp2(a1, w2, b2, g2, be2):
    return pl.pallas_call(
        _mlp2_body,
        grid=(HS // TJ2,),
        in_specs=[
            pl.BlockSpec((B, HS), lambda j: (0, 0)),
            pl.BlockSpec((TJ2, HS), lambda j: (j, 0)),
            pl.BlockSpec((1, TJ2), lambda j: (0, j)),
            pl.BlockSpec((1, TJ2), lambda j: (0, j)),
            pl.BlockSpec((1, TJ2), lambda j: (0, j)),
        ],
        out_specs=pl.BlockSpec((B, TJ2), lambda j: (0, j)),
        out_shape=jax.ShapeDtypeStruct((B, HS), jnp.float32),
        compiler_params=pltpu.CompilerParams(
            dimension_semantics=("arbitrary",),
        ),
    )(a1, w2, b2, g2, be2)


def kernel(x, u, batch, W1, b1, g1, be1, W2, b2, g2, be2):
    bat = batch.astype(jnp.int32)
    pooled = _pool_fn()(x, bat)
    a1 = _mlp1(
        u, pooled, W1,
        b1.reshape(1, HS), g1.reshape(1, HS), be1.reshape(1, HS),
    )
    return _mlp2(
        a1, W2, b2.reshape(1, HS), g2.reshape(1, HS), be2.reshape(1, HS)
    )
